# baseline (speedup 1.0000x reference)
import jax
import jax.numpy as jnp
from jax import lax
from jax.experimental import pallas as pl
from jax.experimental.pallas import tpu as pltpu
from jax.experimental.pallas import tpu_sc as plsc

_TEMP = 2.0 / 3.0
_MAGIC = 0.8
_R, _N = 32, 32768
_KZ = _N - 16384
_L = 16
_U = 8
_GW = _L * _U
_GROUPS = _N // _GW


def _row_body(s_hbm, out_hbm, row_v, egrp_v, sem_in0, sem_in1, sem_in2,
              sem_in3, sem_out):
    wid = lax.axis_index("c") * 16 + lax.axis_index("s")
    iota16 = lax.iota(jnp.int32, _L)

    _QE = _N // 4
    in_sems = (sem_in0, sem_in1, sem_in2, sem_in3)
    in_descs = [
        pltpu.async_copy(s_hbm.at[wid, pl.ds(q * _QE, _QE)],
                         row_v.at[pl.ds(q * _QE, _QE)], in_sems[q])
        for q in range(4)
    ]

    def load_bits(base):
        return lax.bitcast_convert_type(row_v[pl.ds(base, _L)], jnp.int32)

    def sig(v):
        t = v / jnp.float32(_TEMP) * jnp.float32(_MAGIC)
        return 1.0 / (1.0 + jnp.exp(-t))

    in_descs[0].wait()
    s0 = sig(row_v[pl.ds(0, _L)])
    b0 = lax.bitcast_convert_type(s0, jnp.int32)
    sk, _ = plsc.sort_key_val(b0, b0)
    pivot0 = sk[_L // 2 - 1]

    mn = jnp.full((_L,), jnp.int32(2**31 - 1))
    mx = jnp.full((_L,), jnp.int32(-(2**31)))
    ac = jnp.zeros((_L,), jnp.int32)
    for q in range(4):
        if q > 0:
            in_descs[q].wait()

        @plsc.parallel_loop(q * (_QE // _L), (q + 1) * (_QE // _L),
                            unroll=_U, carry=(mn, mx, ac))
        def mm_carry(i, carry):
            mn_, mx_, ac_ = carry
            s = sig(row_v[pl.ds(i * _L, _L)])
            row_v[pl.ds(i * _L, _L)] = s
            b = lax.bitcast_convert_type(s, jnp.int32)
            return (jnp.minimum(mn_, b), jnp.maximum(mx_, b),
                    ac_ + plsc.all_reduce_population_count(b <= pivot0))

        mn, mx, ac = mm_carry

    c0 = ac[0]
    lo0 = mn[0]
    for i in range(1, _L):
        lo0 = jnp.minimum(lo0, mn[i])
    hi0 = mx[0]
    for i in range(1, _L):
        hi0 = jnp.maximum(hi0, mx[i])

    def count_le(t):
        @plsc.parallel_loop(0, _N // _L, unroll=_U,
                            carry=jnp.zeros((_L,), jnp.int32))
        def acc(i, a):
            return a + plsc.all_reduce_population_count(
                load_bits(i * _L) <= t)
        return acc[0]

    def bs_cond(st):
        return st[0] < st[1]

    def bs_body(st):
        lo, hi, c_lo, c_hi, it = st
        span_v = jnp.full((_L,), (hi - lo + 1).astype(jnp.float32))
        num_v = jnp.full((_L,), (_KZ - c_lo).astype(jnp.float32))
        den_v = jnp.full((_L,), (c_hi - c_lo).astype(jnp.float32))
        interp_p = lo - 1 + (num_v / den_v * span_v).astype(jnp.int32)[0]
        bisect_p = (lo + hi) >> 1
        p = jnp.where(lax.rem(it, jnp.int32(6)) == 5, bisect_p, interp_p)
        p = jnp.clip(p, lo, hi - 1)
        c = count_le(p)
        ge = c >= _KZ
        return (jnp.where(ge, lo, p + 1), jnp.where(ge, p, hi),
                jnp.where(ge, c_lo, c), jnp.where(ge, c, c_hi), it + 1)

    ge0 = c0 >= _KZ
    init = (jnp.where(ge0, lo0, pivot0 + 1), jnp.where(ge0, pivot0, hi0),
            jnp.where(ge0, jnp.int32(0), c0),
            jnp.where(ge0, c0, jnp.int32(_N)), jnp.int32(0))
    T, _, cnt_less, _, _ = lax.while_loop(bs_cond, bs_body, init)
    r = _KZ - cnt_less

    @plsc.parallel_loop(0, _GROUPS, unroll=2)
    def eq_body(g):
        base = g * _GW
        cnt_eq = jnp.zeros((_L,), jnp.int32)
        for u in range(_U):
            b = load_bits(base + u * _L)
            cnt_eq = cnt_eq + plsc.all_reduce_population_count(b == T)
        egrp_v[pl.ds(g * _L, _L)] = cnt_eq

    def gscan(g, carry):
        acc, G, rp = carry
        c = egrp_v[pl.ds(g * _L, _L)][0]
        take = (G < 0) & (acc + c >= r)
        G = jnp.where(take, g, G)
        rp = jnp.where(take, r - acc, rp)
        return acc + c, G, rp

    _, G, rp = lax.fori_loop(
        0, _GROUPS, gscan, (jnp.int32(0), jnp.int32(-1), jnp.int32(0)))

    rp_v = jnp.full((_L,), rp, jnp.int32)
    prefix = jnp.zeros((_L,), jnp.int32)
    P_v = jnp.full((_L,), jnp.int32(-1))
    gbase = G * _GW
    for u in range(_U):
        b = load_bits(gbase + u * _L)
        eq = b == T
        cu = plsc.all_reduce_population_count(eq)
        found_here = (P_v < 0) & (prefix + cu >= rp_v)
        rpp = rp_v - prefix
        eq_work = eq
        for t in range(_L - 1):
            more = jnp.full((_L,), jnp.int32(t + 1)) < rpp
            f = plsc.all_reduce_ffs(eq_work)
            eq_work = eq_work & ~(more & (iota16 == f))
        P_here = plsc.all_reduce_ffs(eq_work) + (gbase + u * _L)
        P_v = jnp.where(found_here, P_here, P_v)
        prefix = prefix + cu

    out_descs = []
    for q in range(4):

        @plsc.parallel_loop(q * (_QE // _L), (q + 1) * (_QE // _L),
                            unroll=_U)
        def mask_body(i):
            base = i * _L
            v = row_v[pl.ds(base, _L)]
            b = lax.bitcast_convert_type(v, jnp.int32)
            idx = iota16 + base
            zero = (b < T) | ((b == T) & (idx <= P_v))
            row_v[pl.ds(base, _L)] = jnp.where(zero, 0.0, v)

        out_descs.append(
            pltpu.async_copy(row_v.at[pl.ds(q * _QE, _QE)],
                             out_hbm.at[wid, pl.ds(q * _QE, _QE)], sem_out))
    for d in out_descs:
        d.wait()


def _sc_select(s):
    kfn = pl.kernel(
        _row_body,
        out_type=jax.ShapeDtypeStruct((_R, _N), jnp.float32),
        mesh=plsc.VectorSubcoreMesh(
            core_axis_name="c", subcore_axis_name="s",
            num_cores=2, num_subcores=16),
        scratch_types=[
            pltpu.VMEM((_N,), jnp.float32),
            pltpu.VMEM((_GROUPS * _L,), jnp.int32),
            pltpu.SemaphoreType.DMA,
            pltpu.SemaphoreType.DMA,
            pltpu.SemaphoreType.DMA,
            pltpu.SemaphoreType.DMA,
            pltpu.SemaphoreType.DMA,
        ],
        compiler_params=pltpu.CompilerParams(needs_layout_passes=False),
    )
    return kfn(s)


def kernel(z_loga):
    z2 = z_loga.reshape(-1, z_loga.shape[-1])
    return _sc_select(z2).reshape(_R, _N)


# speedup vs baseline: 1.0889x; 1.0889x over previous
import jax
import jax.numpy as jnp
from jax import lax
from jax.experimental import pallas as pl
from jax.experimental.pallas import tpu as pltpu
from jax.experimental.pallas import tpu_sc as plsc

_TEMP = 2.0 / 3.0
_MAGIC = 0.8
_R, _N = 32, 32768
_KZ = _N - 16384
_L = 16
_U = 8
_GW = _L * _U
_GROUPS = _N // _GW


def _row_body(s_hbm, out_hbm, row_v, egrp_v, sem_in0, sem_in1, sem_in2,
              sem_in3, sem_in4, sem_in5, sem_in6, sem_in7, sem_out):
    wid = lax.axis_index("c") * 16 + lax.axis_index("s")
    iota16 = lax.iota(jnp.int32, _L)

    _SE = _N // 8
    _QE = _N // 4
    in_sems = (sem_in0, sem_in1, sem_in2, sem_in3, sem_in4, sem_in5,
               sem_in6, sem_in7)
    in_descs = [
        pltpu.async_copy(s_hbm.at[wid, pl.ds(q * _SE, _SE)],
                         row_v.at[pl.ds(q * _SE, _SE)], in_sems[q])
        for q in range(8)
    ]

    def load_bits(base):
        return lax.bitcast_convert_type(row_v[pl.ds(base, _L)], jnp.int32)

    in_descs[0].wait()
    b0 = load_bits(0)
    sk, _ = plsc.sort_key_val(b0, b0)
    pivot0 = sk[_L // 2 - 1]

    mn = jnp.full((_L,), jnp.int32(2**31 - 1))
    mx = jnp.full((_L,), jnp.int32(-(2**31)))
    ac = jnp.zeros((_L,), jnp.int32)
    for q in range(8):
        if q > 0:
            in_descs[q].wait()

        @plsc.parallel_loop(q * (_SE // _L), (q + 1) * (_SE // _L),
                            unroll=_U, carry=(mn, mx, ac))
        def mm_carry(i, carry):
            mn_, mx_, ac_ = carry
            b = load_bits(i * _L)
            return (jnp.minimum(mn_, b), jnp.maximum(mx_, b),
                    ac_ + plsc.all_reduce_population_count(b <= pivot0))

        mn, mx, ac = mm_carry

    c0 = ac[0]
    lo0 = mn[0]
    for i in range(1, _L):
        lo0 = jnp.minimum(lo0, mn[i])
    hi0 = mx[0]
    for i in range(1, _L):
        hi0 = jnp.maximum(hi0, mx[i])

    def count_le(t):
        @plsc.parallel_loop(0, _N // _L, unroll=_U,
                            carry=jnp.zeros((_L,), jnp.int32))
        def acc(i, a):
            return a + plsc.all_reduce_population_count(
                load_bits(i * _L) <= t)
        return acc[0]

    def bs_cond(st):
        return st[0] < st[1]

    def bs_body(st):
        lo, hi, c_lo, c_hi, it = st
        span_v = jnp.full((_L,), (hi - lo + 1).astype(jnp.float32))
        num_v = jnp.full((_L,), (_KZ - c_lo).astype(jnp.float32))
        den_v = jnp.full((_L,), (c_hi - c_lo).astype(jnp.float32))
        interp_p = lo - 1 + (num_v / den_v * span_v).astype(jnp.int32)[0]
        bisect_p = (lo + hi) >> 1
        p = jnp.where(lax.rem(it, jnp.int32(6)) == 5, bisect_p, interp_p)
        p = jnp.clip(p, lo, hi - 1)
        c = count_le(p)
        ge = c >= _KZ
        return (jnp.where(ge, lo, p + 1), jnp.where(ge, p, hi),
                jnp.where(ge, c_lo, c), jnp.where(ge, c, c_hi), it + 1)

    ge0 = c0 >= _KZ
    init = (jnp.where(ge0, lo0, pivot0 + 1), jnp.where(ge0, pivot0, hi0),
            jnp.where(ge0, jnp.int32(0), c0),
            jnp.where(ge0, c0, jnp.int32(_N)), jnp.int32(0))
    T, _, cnt_less, _, _ = lax.while_loop(bs_cond, bs_body, init)
    r = _KZ - cnt_less

    @plsc.parallel_loop(0, _GROUPS, unroll=2)
    def eq_body(g):
        base = g * _GW
        cnt_eq = jnp.zeros((_L,), jnp.int32)
        for u in range(_U):
            b = load_bits(base + u * _L)
            cnt_eq = cnt_eq + plsc.all_reduce_population_count(b == T)
        egrp_v[pl.ds(g * _L, _L)] = cnt_eq

    def gscan(g, carry):
        acc, G, rp = carry
        c = egrp_v[pl.ds(g * _L, _L)][0]
        take = (G < 0) & (acc + c >= r)
        G = jnp.where(take, g, G)
        rp = jnp.where(take, r - acc, rp)
        return acc + c, G, rp

    _, G, rp = lax.fori_loop(
        0, _GROUPS, gscan, (jnp.int32(0), jnp.int32(-1), jnp.int32(0)))

    rp_v = jnp.full((_L,), rp, jnp.int32)
    prefix = jnp.zeros((_L,), jnp.int32)
    P_v = jnp.full((_L,), jnp.int32(-1))
    gbase = G * _GW
    for u in range(_U):
        b = load_bits(gbase + u * _L)
        eq = b == T
        cu = plsc.all_reduce_population_count(eq)
        found_here = (P_v < 0) & (prefix + cu >= rp_v)
        rpp = rp_v - prefix
        eq_work = eq
        for t in range(_L - 1):
            more = jnp.full((_L,), jnp.int32(t + 1)) < rpp
            f = plsc.all_reduce_ffs(eq_work)
            eq_work = eq_work & ~(more & (iota16 == f))
        P_here = plsc.all_reduce_ffs(eq_work) + (gbase + u * _L)
        P_v = jnp.where(found_here, P_here, P_v)
        prefix = prefix + cu

    Pc = lax.shift_right_logical(P_v[0], 4)
    out_descs = []
    for q in range(4):
        qlo, qhi = q * (_QE // _L), (q + 1) * (_QE // _L)
        before = Pc >= qhi
        after = Pc < qlo

        @pl.when(before)
        def _():
            @plsc.parallel_loop(qlo, qhi, unroll=_U)
            def mask_le(i):
                v = row_v[pl.ds(i * _L, _L)]
                b = lax.bitcast_convert_type(v, jnp.int32)
                row_v[pl.ds(i * _L, _L)] = jnp.where(b <= T, 0.0, v)

        @pl.when(after)
        def _():
            @plsc.parallel_loop(qlo, qhi, unroll=_U)
            def mask_lt(i):
                v = row_v[pl.ds(i * _L, _L)]
                b = lax.bitcast_convert_type(v, jnp.int32)
                row_v[pl.ds(i * _L, _L)] = jnp.where(b < T, 0.0, v)

        @pl.when(jnp.logical_not(before | after))
        def _():
            @plsc.parallel_loop(qlo, qhi, unroll=_U)
            def mask_full(i):
                base = i * _L
                v = row_v[pl.ds(base, _L)]
                b = lax.bitcast_convert_type(v, jnp.int32)
                idx = iota16 + base
                zero = (b < T) | ((b == T) & (idx <= P_v))
                row_v[pl.ds(base, _L)] = jnp.where(zero, 0.0, v)

        out_descs.append(
            pltpu.async_copy(row_v.at[pl.ds(q * _QE, _QE)],
                             out_hbm.at[wid, pl.ds(q * _QE, _QE)], sem_out))
    for d in out_descs:
        d.wait()


def _sc_select(s):
    kfn = pl.kernel(
        _row_body,
        out_type=jax.ShapeDtypeStruct((_R, _N), jnp.float32),
        mesh=plsc.VectorSubcoreMesh(
            core_axis_name="c", subcore_axis_name="s",
            num_cores=2, num_subcores=16),
        scratch_types=[
            pltpu.VMEM((_N,), jnp.float32),
            pltpu.VMEM((_GROUPS * _L,), jnp.int32),
            pltpu.SemaphoreType.DMA,
            pltpu.SemaphoreType.DMA,
            pltpu.SemaphoreType.DMA,
            pltpu.SemaphoreType.DMA,
            pltpu.SemaphoreType.DMA,
            pltpu.SemaphoreType.DMA,
            pltpu.SemaphoreType.DMA,
            pltpu.SemaphoreType.DMA,
            pltpu.SemaphoreType.DMA,
        ],
        compiler_params=pltpu.CompilerParams(needs_layout_passes=False),
    )
    return kfn(s)


def kernel(z_loga):
    z2 = z_loga.reshape(-1, z_loga.shape[-1])
    s = jax.nn.sigmoid(z2 / _TEMP * _MAGIC)
    return _sc_select(s).reshape(_R, _N)
